# Initial kernel scaffold; baseline (speedup 1.0000x reference)
import jax
import jax.numpy as jnp
from jax.experimental import pallas as pl


def kernel(user_emb, item_emb, A_index, A_weight, A2_index, A2_weight, S1_index, S1_weight, W_i, W_c, W_s):
    raise NotImplementedError("write your pallas kernel here")


# speedup vs baseline: 2.6149x; 2.6149x over previous
import functools

import jax
import jax.numpy as jnp
from jax import lax
from jax.experimental import pallas as pl
from jax.experimental.pallas import tpu as pltpu
from jax.experimental.pallas import tpu_sc as plsc

_U = 5000
_I = 5000
_N = _U + _I
_D = 128
_EA = 320000
_ES = 160000
_L = 3

_NC = 2
_NS = 16
_NW = _NC * _NS
_LANES = 16


def _round_up(x, m):
    return (x + m - 1) // m * m


@functools.lru_cache(maxsize=None)
def _make_spmm(n_in, n_out_pad, n_edges, chunk):
    e_per_tile = n_edges // _NW
    assert e_per_tile * _NW == n_edges
    n_chunks = e_per_tile // chunk
    assert n_chunks * chunk == e_per_tile
    assert chunk % 8 == 0 and chunk <= 128
    rows_per_tile = n_out_pad // _NS
    assert rows_per_tile * _NS == n_out_pad

    mesh = plsc.VectorSubcoreMesh(core_axis_name="c", subcore_axis_name="s")

    @functools.partial(
        pl.kernel,
        out_type=jax.ShapeDtypeStruct((_NC, n_out_pad, _D), jnp.float32),
        mesh=mesh,
        scratch_types=[
            pltpu.VMEM((chunk,), jnp.int32),
            pltpu.VMEM((chunk,), jnp.int32),
            pltpu.VMEM((chunk,), jnp.float32),
            pltpu.VMEM((chunk, _D), jnp.float32),
            pltpu.VMEM_SHARED((n_out_pad, _D), jnp.float32),
            pltpu.SemaphoreType.DMA,
        ],
    )
    def spmm(x_hbm, src_hbm, dst_hbm, w_hbm, zeros_hbm, out_hbm,
             src_v, dst_v, w_v, rows_v, acc_sh, sem):
        c = lax.axis_index("c")
        s = lax.axis_index("s")

        r0 = s * rows_per_tile
        pltpu.sync_copy(zeros_hbm.at[pl.ds(r0, rows_per_tile)],
                        acc_sh.at[pl.ds(r0, rows_per_tile)])
        plsc.subcore_barrier()

        tile_base = (c * _NS + s) * e_per_tile

        def chunk_body(i, carry):
            off = tile_base + i * chunk
            pltpu.sync_copy(src_hbm.at[pl.ds(off, chunk)], src_v)
            pltpu.sync_copy(dst_hbm.at[pl.ds(off, chunk)], dst_v)
            pltpu.sync_copy(w_hbm.at[pl.ds(off, chunk)], w_v)
            pltpu.async_copy(x_hbm.at[src_v], rows_v, sem).wait()

            def row_body(b, carry2):
                w16 = w_v[pl.ds(b * _LANES, _LANES)]
                for j in range(_LANES):
                    r = b * _LANES + j
                    wj = w16[j]
                    for g in range(_D // _LANES):
                        sl = pl.ds(g * _LANES, _LANES)
                        rows_v[r, sl] = rows_v[r, sl] * wj
                return carry2

            lax.fori_loop(0, chunk // _LANES, row_body, 0)
            pltpu.sync_copy(rows_v, acc_sh.at[dst_v], add=True)
            return carry

        lax.fori_loop(0, n_chunks, chunk_body, 0)
        plsc.subcore_barrier()

        pltpu.sync_copy(acc_sh.at[pl.ds(r0, rows_per_tile)],
                        out_hbm.at[c, pl.ds(r0, rows_per_tile)])

    return spmm


@functools.lru_cache(maxsize=None)
def _make_dense(u_pad, final):
    def body(pa_ref, ps_ref, acc_ref, wi_ref, wc_ref, ws_ref,
             all_next_ref, users_soc_ref, acc_out_ref):
        inter = pa_ref[0, :_N] + pa_ref[1, :_N]
        u_int = inter[:_U]
        items_next = inter[_U:]
        users_next = jnp.tanh(
            jnp.dot(u_int, wi_ref[...], preferred_element_type=jnp.float32))
        soc = ps_ref[0, :_U] + ps_ref[1, :_U]
        users_soc = jnp.tanh(
            jnp.dot(soc, wc_ref[...], preferred_element_type=jnp.float32))
        users = (jnp.dot(users_next, ws_ref[:_D],
                         preferred_element_type=jnp.float32) +
                 jnp.dot(users_soc, ws_ref[_D:],
                         preferred_element_type=jnp.float32))
        users = users / jnp.sqrt(jnp.sum(users * users))
        all_next_ref[:_U] = users_next
        all_next_ref[_U:] = items_next
        users_soc_ref[...] = users_soc
        new_acc = acc_ref[...] + jnp.concatenate([users, items_next], axis=0)
        if final:
            new_acc = new_acc * (1.0 / (_L + 1))
        acc_out_ref[...] = new_acc

    return pl.pallas_call(
        body,
        out_shape=[
            jax.ShapeDtypeStruct((_N, _D), jnp.float32),
            jax.ShapeDtypeStruct((_U, _D), jnp.float32),
            jax.ShapeDtypeStruct((_N, _D), jnp.float32),
        ],
    )


def kernel(user_emb, item_emb, A_index, A_weight, A2_index, A2_weight,
           S1_index, S1_weight, W_i, W_c, W_s):
    n_pad = _round_up(_N, 8 * _NS)
    u_pad = _round_up(_U, 8 * _NS)
    all_emb = jnp.concatenate([user_emb, item_emb], axis=0)
    users_emb = user_emb
    acc = all_emb
    zeros_n = jnp.zeros((n_pad, _D), jnp.float32)
    zeros_u = jnp.zeros((u_pad, _D), jnp.float32)

    spmm_a = _make_spmm(_N, n_pad, _EA, 80)
    spmm_s = _make_spmm(_U, u_pad, _ES, 40)

    for layer in range(_L):
        idx, w = (A_index, A_weight) if layer == 0 else (A2_index, A2_weight)
        pa = spmm_a(all_emb, idx[1], idx[0], w, zeros_n)
        ps = spmm_s(users_emb, S1_index[1], S1_index[0], S1_weight, zeros_u)
        dense = _make_dense(u_pad, layer == _L - 1)
        all_emb, users_emb, acc = dense(pa, ps, acc, W_i, W_c, W_s)

    return acc[:_U], acc[_U:]


# speedup vs baseline: 3.0039x; 1.1488x over previous
import functools

import jax
import jax.numpy as jnp
from jax import lax
from jax.experimental import pallas as pl
from jax.experimental.pallas import tpu as pltpu
from jax.experimental.pallas import tpu_sc as plsc

_U = 5000
_I = 5000
_N = _U + _I
_D = 128
_EA = 320000
_ES = 160000
_L = 3

_NC = 2
_NS = 16
_NW = _NC * _NS
_LANES = 16


def _round_up(x, m):
    return (x + m - 1) // m * m


_CH = 128


@functools.lru_cache(maxsize=None)
def _make_spmm(n_in, n_out_pad, n_chunks):
    rows_per_tile = n_out_pad // _NS
    assert rows_per_tile * _NS == n_out_pad and rows_per_tile % 8 == 0
    assert n_chunks % 2 == 0

    mesh = plsc.VectorSubcoreMesh(core_axis_name="c", subcore_axis_name="s")
    e_tile = n_chunks * _CH

    @functools.partial(
        pl.kernel,
        out_type=jax.ShapeDtypeStruct((_NC, n_out_pad, _D), jnp.float32),
        mesh=mesh,
        scratch_types=[
            pltpu.VMEM((e_tile,), jnp.int32),
            pltpu.VMEM((_CH,), jnp.int32),
            pltpu.VMEM((_CH,), jnp.int32),
            pltpu.VMEM((_CH,), jnp.float32),
            pltpu.VMEM((_CH,), jnp.float32),
            pltpu.VMEM((_CH, _D), jnp.float32),
            pltpu.VMEM((_CH, _D), jnp.float32),
            pltpu.VMEM_SHARED((n_out_pad, _D), jnp.float32),
            pltpu.SemaphoreType.DMA,
            pltpu.SemaphoreType.DMA,
            pltpu.SemaphoreType.DMA,
            pltpu.SemaphoreType.DMA,
            pltpu.SemaphoreType.DMA,
            pltpu.SemaphoreType.DMA,
            pltpu.SemaphoreType.DMA,
        ],
    )
    def spmm(x_hbm, src_hbm, dst_hbm, w_hbm, zeros_hbm, out_hbm,
             src_v, dstb0, dstb1, wb0, wb1, rows0, rows1, acc_sh,
             sem_i, sem0, sem1, semd0, semd1, semw0, semw1):
        c = lax.axis_index("c")
        s = lax.axis_index("s")
        wid = c * _NS + s

        e0 = wid * e_tile
        cp_s = pltpu.async_copy(src_hbm.at[pl.ds(e0, e_tile)], src_v, sem_i)
        r0 = s * rows_per_tile
        pltpu.sync_copy(zeros_hbm.at[pl.ds(r0, rows_per_tile)],
                        acc_sh.at[pl.ds(r0, rows_per_tile)])
        cp_s.wait()
        plsc.subcore_barrier()

        def start_gather(i, buf, sem):
            pltpu.async_copy(x_hbm.at[src_v.at[pl.ds(i * _CH, _CH)]], buf, sem)

        def wait_gather(i, buf, sem):
            pltpu.make_async_copy(
                x_hbm.at[src_v.at[pl.ds(i * _CH, _CH)]], buf, sem).wait()

        def start_dw(i, dbuf, wbuf, semd, semw):
            pltpu.async_copy(dst_hbm.at[pl.ds(e0 + i * _CH, _CH)], dbuf, semd)
            pltpu.async_copy(w_hbm.at[pl.ds(e0 + i * _CH, _CH)], wbuf, semw)

        def wait_dw(i, dbuf, wbuf, semd, semw):
            pltpu.make_async_copy(
                dst_hbm.at[pl.ds(e0 + i * _CH, _CH)], dbuf, semd).wait()
            pltpu.make_async_copy(
                w_hbm.at[pl.ds(e0 + i * _CH, _CH)], wbuf, semw).wait()

        def scale(buf, wbuf):
            def block_body(b, carry):
                w16 = wbuf[pl.ds(b * _LANES, _LANES)]
                for j in range(_LANES):
                    r = b * _LANES + j
                    wj = w16[j]
                    for g in range(_D // _LANES):
                        sl = pl.ds(g * _LANES, _LANES)
                        buf[r, sl] = buf[r, sl] * wj
                return carry

            lax.fori_loop(0, _CH // _LANES, block_body, 0)

        start_gather(0, rows0, sem0)
        start_dw(0, dstb0, wb0, semd0, semw0)

        def pair_body(k, carry):
            i0 = 2 * k
            start_gather(i0 + 1, rows1, sem1)
            start_dw(i0 + 1, dstb1, wb1, semd1, semw1)
            wait_gather(i0, rows0, sem0)
            wait_dw(i0, dstb0, wb0, semd0, semw0)
            scale(rows0, wb0)
            pltpu.sync_copy(rows0, acc_sh.at[dstb0], add=True)

            @pl.when(i0 + 2 < n_chunks)
            def _():
                start_gather(i0 + 2, rows0, sem0)
                start_dw(i0 + 2, dstb0, wb0, semd0, semw0)

            wait_gather(i0 + 1, rows1, sem1)
            wait_dw(i0 + 1, dstb1, wb1, semd1, semw1)
            scale(rows1, wb1)
            pltpu.sync_copy(rows1, acc_sh.at[dstb1], add=True)
            return carry

        lax.fori_loop(0, n_chunks // 2, pair_body, 0)
        plsc.subcore_barrier()

        pltpu.sync_copy(acc_sh.at[pl.ds(r0, rows_per_tile)],
                        out_hbm.at[c, pl.ds(r0, rows_per_tile)])

    return spmm


def _shard_edges(idx, w, n_edges):
    e_per_tile = n_edges // _NW
    e_pad = _round_up(e_per_tile, 2 * _CH)
    n_chunks = e_pad // _CH
    pad = [(0, 0), (0, e_pad - e_per_tile)]
    dst = jnp.pad(idx[0].reshape(_NW, e_per_tile), pad).reshape(-1)
    src = jnp.pad(idx[1].reshape(_NW, e_per_tile), pad).reshape(-1)
    wp = jnp.pad(w.reshape(_NW, e_per_tile), pad).reshape(-1)
    return src, dst, wp, n_chunks


@functools.lru_cache(maxsize=None)
def _make_dense(u_pad, final):
    def body(pa_ref, ps_ref, acc_ref, wi_ref, wc_ref, ws_ref,
             all_next_ref, users_soc_ref, acc_out_ref):
        inter = pa_ref[0, :_N] + pa_ref[1, :_N]
        u_int = inter[:_U]
        items_next = inter[_U:]
        users_next = jnp.tanh(
            jnp.dot(u_int, wi_ref[...], preferred_element_type=jnp.float32))
        soc = ps_ref[0, :_U] + ps_ref[1, :_U]
        users_soc = jnp.tanh(
            jnp.dot(soc, wc_ref[...], preferred_element_type=jnp.float32))
        users = (jnp.dot(users_next, ws_ref[:_D],
                         preferred_element_type=jnp.float32) +
                 jnp.dot(users_soc, ws_ref[_D:],
                         preferred_element_type=jnp.float32))
        users = users / jnp.sqrt(jnp.sum(users * users))
        all_next_ref[:_U] = users_next
        all_next_ref[_U:] = items_next
        users_soc_ref[...] = users_soc
        new_acc = acc_ref[...] + jnp.concatenate([users, items_next], axis=0)
        if final:
            new_acc = new_acc * (1.0 / (_L + 1))
        acc_out_ref[...] = new_acc

    return pl.pallas_call(
        body,
        out_shape=[
            jax.ShapeDtypeStruct((_N, _D), jnp.float32),
            jax.ShapeDtypeStruct((_U, _D), jnp.float32),
            jax.ShapeDtypeStruct((_N, _D), jnp.float32),
        ],
    )


def kernel(user_emb, item_emb, A_index, A_weight, A2_index, A2_weight,
           S1_index, S1_weight, W_i, W_c, W_s):
    n_pad = _round_up(_N, 8 * _NS)
    u_pad = _round_up(_U, 8 * _NS)
    all_emb = jnp.concatenate([user_emb, item_emb], axis=0)
    users_emb = user_emb
    acc = all_emb
    zeros_n = jnp.zeros((n_pad, _D), jnp.float32)
    zeros_u = jnp.zeros((u_pad, _D), jnp.float32)

    a1_src, a1_dst, a1_w, nch_a = _shard_edges(A_index, A_weight, _EA)
    a2_src, a2_dst, a2_w, _ = _shard_edges(A2_index, A2_weight, _EA)
    s_src, s_dst, s_w, nch_s = _shard_edges(S1_index, S1_weight, _ES)

    spmm_a = _make_spmm(_N, n_pad, nch_a)
    spmm_s = _make_spmm(_U, u_pad, nch_s)

    for layer in range(_L):
        src, dst, w = ((a1_src, a1_dst, a1_w) if layer == 0 else
                       (a2_src, a2_dst, a2_w))
        pa = spmm_a(all_emb, src, dst, w, zeros_n)
        ps = spmm_s(users_emb, s_src, s_dst, s_w, zeros_u)
        dense = _make_dense(u_pad, layer == _L - 1)
        all_emb, users_emb, acc = dense(pa, ps, acc, W_i, W_c, W_s)

    return acc[:_U], acc[_U:]
